# baseline (speedup 1.0000x reference)
import functools

import jax
import jax.numpy as jnp
from jax import lax
from jax.experimental import pallas as pl
from jax.experimental.pallas import tpu as pltpu
from jax.experimental.pallas import tpu_sc as plsc

BATCH = 4096
MAX_LEN = 200
EMB = 32
NC = 2
NS = 16
NW = NC * NS
BPW = BATCH // NW
PITCH = EMB + 1
UNROLL = 4


def _body(
    idxT_hbm, table_hbm, pos_hbm, out_hbm, idx_v, buf, skew, blk, pos_v, sem
):
    wid = lax.axis_index("s") * NC + lax.axis_index("c")
    pltpu.sync_copy(idxT_hbm.at[:, pl.ds(wid * BPW, BPW)], idx_v)
    pltpu.sync_copy(pos_hbm, pos_v)

    iota16 = lax.iota(jnp.int32, 16)
    iota33 = iota16 * PITCH

    def pos_body(l, carry):
        pltpu.async_copy(table_hbm.at[idx_v.at[l]], buf, sem).wait()
        pos_h = [pos_v[l, pl.ds(16 * h, 16)] for h in range(2)]

        def tok_body(b4, carry2):
            for u in range(UNROLL):
                b = b4 * UNROLL + u
                for h in range(2):
                    x = buf[b, pl.ds(16 * h, 16)] + pos_h[h]
                    plsc.store_scatter(
                        skew, [iota16 + (b * PITCH + 16 * h)], x
                    )
            return carry2

        lax.fori_loop(0, BPW // UNROLL, tok_body, 0)
        for g in range(BPW // 16):
            base_g = 16 * PITCH * g
            for f in range(EMB):
                x = plsc.load_gather(skew, [iota33 + (base_g + f)])
                blk[pl.ds(((f >> 3) << 10) + ((f & 7) << 7) + 16 * g, 16)] = x
        for fh in range(EMB // 8):
            pltpu.sync_copy(
                blk.at[pl.ds(fh * 8 * BPW, 8 * BPW)],
                out_hbm.at[l].at[fh].at[wid],
            )
        return carry

    lax.fori_loop(0, MAX_LEN, pos_body, 0)


@jax.jit
def _run(idxT, token_table, pos_table):
    mesh = plsc.VectorSubcoreMesh(core_axis_name="c", subcore_axis_name="s")
    k = functools.partial(
        pl.kernel,
        mesh=mesh,
        out_type=jax.ShapeDtypeStruct(
            (MAX_LEN, EMB // 8, NW, 8 * BPW), jnp.float32
        ),
        scratch_types=[
            pltpu.VMEM((MAX_LEN, BPW), jnp.int32),
            pltpu.VMEM((BPW, EMB), jnp.float32),
            pltpu.VMEM((BPW * PITCH,), jnp.float32),
            pltpu.VMEM((EMB * BPW,), jnp.float32),
            pltpu.VMEM((MAX_LEN, EMB), jnp.float32),
            pltpu.SemaphoreType.DMA,
        ],
        compiler_params=pltpu.CompilerParams(
            use_tc_tiling_on_sc=False, needs_layout_passes=False
        ),
    )(_body)
    return k(idxT, token_table, pos_table)


def kernel(batch_seqs, token_table, pos_table):
    out4 = _run(batch_seqs.T, token_table, pos_table)
    out5 = out4.reshape(MAX_LEN, EMB // 8, NW, 8, BPW)
    return out5.transpose(2, 4, 0, 1, 3).reshape(BATCH, MAX_LEN, EMB)


# speedup vs baseline: 1.1563x; 1.1563x over previous
import functools

import jax
import jax.numpy as jnp
from jax import lax
from jax.experimental import pallas as pl
from jax.experimental.pallas import tpu as pltpu
from jax.experimental.pallas import tpu_sc as plsc

BATCH = 4096
MAX_LEN = 200
EMB = 32
NC = 2
NS = 16
NW = NC * NS
BPW = BATCH // NW
UNROLL = 4


def _body(
    idxT_hbm, table_hbm, pos_hbm, out_hbm, idx_v, bufs, pos_v, sem_a, sem_b
):
    wid = lax.axis_index("s") * NC + lax.axis_index("c")
    pltpu.sync_copy(idxT_hbm.at[:, pl.ds(wid * BPW, BPW)], idx_v)
    pltpu.sync_copy(pos_hbm, pos_v)

    def gather(l, slot, sem):
        pltpu.async_copy(table_hbm.at[idx_v.at[l]], bufs.at[slot], sem)

    def consume(l, slot, sem):
        pltpu.make_async_copy(
            table_hbm.at[idx_v.at[l]], bufs.at[slot], sem
        ).wait()
        pos_h = [pos_v[l, pl.ds(16 * h, 16)] for h in range(2)]

        def tok_body(b4, carry2):
            for u in range(UNROLL):
                b = b4 * UNROLL + u
                for h in range(2):
                    bufs[slot, b, pl.ds(16 * h, 16)] = (
                        bufs[slot, b, pl.ds(16 * h, 16)] + pos_h[h]
                    )
            return carry2

        lax.fori_loop(0, BPW // UNROLL, tok_body, 0)
        pltpu.sync_copy(bufs.at[slot], out_hbm.at[l].at[wid])

    gather(0, 0, sem_a)

    def pos_body(i, carry):
        l0 = 2 * i
        gather(l0 + 1, 1, sem_b)
        consume(l0, 0, sem_a)
        gather(jnp.minimum(l0 + 2, MAX_LEN - 1), 0, sem_a)
        consume(l0 + 1, 1, sem_b)
        return carry

    lax.fori_loop(0, MAX_LEN // 2, pos_body, 0)
    pltpu.make_async_copy(
        table_hbm.at[idx_v.at[MAX_LEN - 1]], bufs.at[0], sem_a
    ).wait()


@jax.jit
def _run(idxT, token_table, pos_table):
    mesh = plsc.VectorSubcoreMesh(core_axis_name="c", subcore_axis_name="s")
    k = functools.partial(
        pl.kernel,
        mesh=mesh,
        out_type=jax.ShapeDtypeStruct((MAX_LEN, NW, BPW, EMB), jnp.float32),
        scratch_types=[
            pltpu.VMEM((MAX_LEN, BPW), jnp.int32),
            pltpu.VMEM((2, BPW, EMB), jnp.float32),
            pltpu.VMEM((MAX_LEN, EMB), jnp.float32),
            pltpu.SemaphoreType.DMA,
            pltpu.SemaphoreType.DMA,
        ],
        compiler_params=pltpu.CompilerParams(
            use_tc_tiling_on_sc=False, needs_layout_passes=False
        ),
    )(_body)
    return k(idxT, token_table, pos_table)


def kernel(batch_seqs, token_table, pos_table):
    out4 = _run(batch_seqs.T, token_table, pos_table)
    return out4.transpose(1, 2, 0, 3).reshape(BATCH, MAX_LEN, EMB)
